# baseline (speedup 1.0000x reference)
import functools

import jax
import jax.numpy as jnp
import numpy as np
from jax import lax
from jax.experimental import pallas as pl
from jax.experimental.pallas import tpu as pltpu
from jax.experimental.pallas import tpu_sc as plsc

B = 16384
P = 64
V = 832
VP = 1024
VPW = VP // 4
T = VPW // 128
VE = 896
H1, H2 = 512, 256

NC, NS = 2, 16
NW = NC * NS
GRP = 16
NBUF = 2

NCHUNK = 4
NB = B // NCHUNK
RW = NB // NW
NGRP = RW // GRP

BLK = 1024


_mesh = plsc.VectorSubcoreMesh(core_axis_name="c", subcore_axis_name="s")


@functools.partial(
    pl.kernel,
    mesh=_mesh,
    compiler_params=pltpu.CompilerParams(use_tc_tiling_on_sc=False,
                                         needs_layout_passes=False),
    out_type=jax.ShapeDtypeStruct((NB * VPW,), jnp.int32),
    scratch_types=[
        pltpu.VMEM((RW, P), jnp.int32),
        pltpu.VMEM((GRP * VPW,), jnp.int32),
        pltpu.VMEM((GRP * VPW,), jnp.int32),
        pltpu.SemaphoreType.DMA,
        pltpu.SemaphoreType.DMA,
    ],
)
def _hist(x_hbm, out_hbm, xv, buf0, buf1, sem0, sem1):
    wid = lax.axis_index("s") * NC + lax.axis_index("c")
    base = wid * RW
    pltpu.sync_copy(x_hbm.at[pl.ds(base, RW), :], xv)

    lanes = lax.broadcasted_iota(jnp.int32, (16,), 0)
    lane_base = (lanes >> 3) * (VPW * 8) + (lanes & 7) * 128
    one = jnp.ones((16,), jnp.int32)
    zeros16 = jnp.zeros((16,), jnp.int32)
    bufs = (buf0, buf1)
    sems = (sem0, sem1)

    def pair_body(gp, carry):
        for bi in range(NBUF):
            g = gp * NBUF + bi
            buf, sem = bufs[bi], sems[bi]

            @pl.when(gp > 0)
            def _wait():
                pltpu.make_async_copy(
                    out_hbm.at[pl.ds(0, GRP * VPW)], buf, sem).wait()

            @plsc.parallel_loop(0, GRP * VPW, 16, unroll=8)
            def _zero(i):
                buf[pl.ds(i, 16)] = zeros16

            grow = g * GRP

            @plsc.parallel_loop(0, P, 1, unroll=8)
            def _scat(p):
                col = plsc.load_gather(
                    xv, [grow + lanes, (lanes + p) & (P - 1)])
                off = lane_base + ((col >> 9) << 10) + (col & 127)
                val = one << (((col >> 7) & 3) << 3)
                plsc.addupdate_scatter(buf, [off], val)

            pltpu.async_copy(
                buf, out_hbm.at[pl.ds((base + g * GRP) * VPW, GRP * VPW)],
                sem)
        return carry

    lax.fori_loop(0, NGRP // NBUF, pair_body, 0)

    for bi in range(NBUF):
        pltpu.make_async_copy(
            out_hbm.at[pl.ds(0, GRP * VPW)], bufs[bi], sems[bi]).wait()


def _e1_body(emb_ref, w1_ref, out_ref):
    out_ref[V:, :] = jnp.zeros((VE - V, H1), jnp.bfloat16)
    out_ref[:V, :] = lax.dot_general(
        emb_ref[...], w1_ref[...], (((1,), (1,)), ((), ())),
        preferred_element_type=jnp.float32).astype(jnp.bfloat16)


_e1_call = pl.pallas_call(
    _e1_body,
    out_shape=jax.ShapeDtypeStruct((VE, H1), jnp.bfloat16),
)


def _mlp_body(c_ref, e1_ref, b1_ref, w2_ref, b2_ref, w3_ref, b3_ref, o_ref):
    c4 = c_ref[...]
    pieces = []
    for j in range(T):
        wj = c4[:, j].reshape(BLK, 128)
        for k in range(4):
            if j * 512 + k * 128 >= VE:
                continue
            pieces.append(((wj >> (8 * k)) & 0xFF).astype(jnp.bfloat16))
    c = jnp.concatenate(pieces, axis=1)
    acc = jnp.dot(c, e1_ref[...], preferred_element_type=jnp.float32)
    h1 = jnp.maximum(acc + b1_ref[...], 0.0)
    h2 = lax.dot_general(h1, w2_ref[...], (((1,), (1,)), ((), ())),
                         preferred_element_type=jnp.float32)
    h2 = jnp.maximum(h2 + b2_ref[...], 0.0)
    o_ref[...] = jnp.sum(h2 * w3_ref[...], axis=1, keepdims=True) + b3_ref[...]


_mlp_call = pl.pallas_call(
    _mlp_body,
    grid=(NB // BLK,),
    in_specs=[
        pl.BlockSpec((BLK // 8, T, 8, 128), lambda i: (i, 0, 0, 0)),
        pl.BlockSpec((VE, H1), lambda i: (0, 0)),
        pl.BlockSpec((1, H1), lambda i: (0, 0)),
        pl.BlockSpec((H2, H1), lambda i: (0, 0)),
        pl.BlockSpec((1, H2), lambda i: (0, 0)),
        pl.BlockSpec((1, H2), lambda i: (0, 0)),
        pl.BlockSpec((1, 1), lambda i: (0, 0)),
    ],
    out_specs=pl.BlockSpec((BLK, 1), lambda i: (i, 0)),
    out_shape=jax.ShapeDtypeStruct((NB, 1), jnp.float32),
)


def kernel(x, emb, W1, b1, W2, b2, W3, b3):
    xi = x.astype(jnp.int32)
    E1 = _e1_call(emb, W1)
    b1r, b2r, b3r = b1.reshape(1, H1), b2.reshape(1, H2), b3.reshape(1, 1)
    outs = []
    for i in range(NCHUNK):
        Cp = _hist(xi[i * NB:(i + 1) * NB])
        outs.append(_mlp_call(Cp.reshape(NB // 8, T, 8, 128), E1,
                              b1r, W2, b2r, W3, b3r))
    return jnp.concatenate(outs, axis=0)


# speedup vs baseline: 1.0897x; 1.0897x over previous
import functools

import jax
import jax.numpy as jnp
from jax import lax
from jax.experimental import pallas as pl
from jax.experimental.pallas import tpu as pltpu
from jax.experimental.pallas import tpu_sc as plsc

B = 16384
P = 64
V = 832
VP = 1024
VPW = VP // 4
T = VPW // 128
VE = 896
H1, H2 = 512, 256

NC, NS = 2, 16
NW = NC * NS
GRP = 16
NBUF = 2

NCHUNK = 2
NB = B // NCHUNK
RW = NB // NW
NGRP = RW // GRP

BLK = 1024


_mesh = plsc.VectorSubcoreMesh(core_axis_name="c", subcore_axis_name="s")


@functools.partial(
    pl.kernel,
    mesh=_mesh,
    compiler_params=pltpu.CompilerParams(use_tc_tiling_on_sc=False,
                                         needs_layout_passes=False),
    out_type=jax.ShapeDtypeStruct((NB * VPW,), jnp.int32),
    scratch_types=[
        pltpu.VMEM((RW, P), jnp.int32),
        pltpu.VMEM((GRP * VPW,), jnp.int32),
        pltpu.VMEM((GRP * VPW,), jnp.int32),
        pltpu.SemaphoreType.DMA,
        pltpu.SemaphoreType.DMA,
    ],
)
def _hist(x_hbm, out_hbm, xv, buf0, buf1, sem0, sem1):
    wid = lax.axis_index("s") * NC + lax.axis_index("c")
    base = wid * RW
    pltpu.sync_copy(x_hbm.at[pl.ds(base, RW), :], xv)

    lanes = lax.broadcasted_iota(jnp.int32, (16,), 0)
    lane_base = (lanes >> 3) * (VPW * 8) + (lanes & 7) * 128
    one = jnp.ones((16,), jnp.int32)
    zeros16 = jnp.zeros((16,), jnp.int32)
    bufs = (buf0, buf1)
    sems = (sem0, sem1)

    def pair_body(gp, carry):
        for bi in range(NBUF):
            g = gp * NBUF + bi
            buf, sem = bufs[bi], sems[bi]

            @pl.when(gp > 0)
            def _wait():
                pltpu.make_async_copy(
                    out_hbm.at[pl.ds(0, GRP * VPW)], buf, sem).wait()

            @plsc.parallel_loop(0, GRP * VPW, 16, unroll=8)
            def _zero(i):
                buf[pl.ds(i, 16)] = zeros16

            grow = g * GRP

            @plsc.parallel_loop(0, P, 1, unroll=8)
            def _scat(p):
                col = plsc.load_gather(
                    xv, [grow + lanes, (lanes + p) & (P - 1)])
                off = lane_base + ((col >> 9) << 10) + (col & 127)
                val = one << (((col >> 7) & 3) << 3)
                plsc.addupdate_scatter(buf, [off], val)

            pltpu.async_copy(
                buf, out_hbm.at[pl.ds((base + g * GRP) * VPW, GRP * VPW)],
                sem)
        return carry

    lax.fori_loop(0, NGRP // NBUF, pair_body, 0)

    for bi in range(NBUF):
        pltpu.make_async_copy(
            out_hbm.at[pl.ds(0, GRP * VPW)], bufs[bi], sems[bi]).wait()


def _e1_body(emb_ref, w1_ref, out_ref):
    out_ref[V:, :] = jnp.zeros((VE - V, H1), jnp.bfloat16)
    out_ref[:V, :] = lax.dot_general(
        emb_ref[...], w1_ref[...], (((1,), (1,)), ((), ())),
        preferred_element_type=jnp.float32).astype(jnp.bfloat16)


_e1_call = pl.pallas_call(
    _e1_body,
    out_shape=jax.ShapeDtypeStruct((VE, H1), jnp.bfloat16),
)


def _mlp_body(c_ref, e1_ref, b1_ref, w2_ref, b2_ref, w3_ref, b3_ref, o_ref):
    c4 = c_ref[...]
    pieces = []
    for j in range(T):
        wj = c4[:, j].reshape(BLK, 128)
        for k in range(4):
            if j * 512 + k * 128 >= VE:
                continue
            pieces.append(((wj >> (8 * k)) & 0xFF).astype(jnp.bfloat16))
    c = jnp.concatenate(pieces, axis=1)
    acc = jnp.dot(c, e1_ref[...], preferred_element_type=jnp.float32)
    h1 = jnp.maximum(acc + b1_ref[...], 0.0)
    h2 = lax.dot_general(h1, w2_ref[...], (((1,), (1,)), ((), ())),
                         preferred_element_type=jnp.float32)
    h2 = jnp.maximum(h2 + b2_ref[...], 0.0)
    o_ref[...] = jnp.sum(h2 * w3_ref[...], axis=1, keepdims=True) + b3_ref[...]


_mlp_call = pl.pallas_call(
    _mlp_body,
    grid=(NB // BLK,),
    in_specs=[
        pl.BlockSpec((BLK // 8, T, 8, 128), lambda i: (i, 0, 0, 0)),
        pl.BlockSpec((VE, H1), lambda i: (0, 0)),
        pl.BlockSpec((1, H1), lambda i: (0, 0)),
        pl.BlockSpec((H2, H1), lambda i: (0, 0)),
        pl.BlockSpec((1, H2), lambda i: (0, 0)),
        pl.BlockSpec((1, H2), lambda i: (0, 0)),
        pl.BlockSpec((1, 1), lambda i: (0, 0)),
    ],
    out_specs=pl.BlockSpec((BLK, 1), lambda i: (i, 0)),
    out_shape=jax.ShapeDtypeStruct((NB, 1), jnp.float32),
)


def kernel(x, emb, W1, b1, W2, b2, W3, b3):
    xi = x.astype(jnp.int32)
    E1 = _e1_call(emb, W1)
    b1r, b2r, b3r = b1.reshape(1, H1), b2.reshape(1, H2), b3.reshape(1, 1)
    outs = []
    for i in range(NCHUNK):
        Cp = _hist(xi[i * NB:(i + 1) * NB])
        outs.append(_mlp_call(Cp.reshape(NB // 8, T, 8, 128), E1,
                              b1r, W2, b2r, W3, b3r))
    return jnp.concatenate(outs, axis=0)
